# initial kernel scaffold (unmeasured)
import jax
import jax.numpy as jnp
from jax import lax
from jax.experimental import pallas as pl
from jax.experimental.pallas import tpu as pltpu

N_DEV = 4
B, SQ, SKV_SH, HQ, DH = 2, 256, 256, 16, 64
H_LOC = HQ // N_DEV
SKV = SKV_SH * N_DEV
D_MODEL = 512


def kernel(x, Wq, K_ext, V_ext, Wo):
    def body(x_ref, wq_ref, k_ref, v_ref, wo_ref, out_ref,
             kf_ref, vf_ref, ar_ref, p_ref,
             k_send, k_recv, v_send, v_recv, ar_send, ar_recv, loc_sem):
        my = lax.axis_index("i")

        bsem = pltpu.get_barrier_semaphore()
        for d in range(1, N_DEV):
            pl.semaphore_signal(
                bsem, inc=1,
                device_id=((my + d) % N_DEV,),
                device_id_type=pl.DeviceIdType.MESH,
            )
        pl.semaphore_wait(bsem, N_DEV - 1)

        kc = pltpu.make_async_copy(
            k_ref.at[:, :, pl.ds(H_LOC * my, H_LOC), :],
            kf_ref.at[:, pl.ds(SKV_SH * my, SKV_SH), :, :],
            loc_sem.at[0],
        )
        vc = pltpu.make_async_copy(
            v_ref.at[:, :, pl.ds(H_LOC * my, H_LOC), :],
            vf_ref.at[:, pl.ds(SKV_SH * my, SKV_SH), :, :],
            loc_sem.at[1],
        )
        kc.start()
        vc.start()

        kv_rdmas = []
        for d in range(1, N_DEV):
            tgt = (my + d) % N_DEV
            kr = pltpu.make_async_remote_copy(
                src_ref=k_ref.at[:, :, pl.ds(H_LOC * tgt, H_LOC), :],
                dst_ref=kf_ref.at[:, pl.ds(SKV_SH * my, SKV_SH), :, :],
                send_sem=k_send.at[tgt],
                recv_sem=k_recv.at[my],
                device_id=(tgt,),
                device_id_type=pl.DeviceIdType.MESH,
            )
            vr = pltpu.make_async_remote_copy(
                src_ref=v_ref.at[:, :, pl.ds(H_LOC * tgt, H_LOC), :],
                dst_ref=vf_ref.at[:, pl.ds(SKV_SH * my, SKV_SH), :, :],
                send_sem=v_send.at[tgt],
                recv_sem=v_recv.at[my],
                device_id=(tgt,),
                device_id_type=pl.DeviceIdType.MESH,
            )
            kr.start()
            vr.start()
            kv_rdmas.append((kr, vr))

        q2d = jnp.dot(
            x_ref[...].reshape(B * SQ, D_MODEL), wq_ref[...],
            preferred_element_type=jnp.float32,
        )
        q = q2d.reshape(B, SQ, H_LOC, DH)

        qi = lax.broadcasted_iota(jnp.int32, (SQ, SKV), 0)
        ki = lax.broadcasted_iota(jnp.int32, (SQ, SKV), 1)
        mask = (jnp.abs(qi - ki) <= 128) | (ki < 32) | (qi < 32)

        kc.wait()
        vc.wait()
        for d in range(1, N_DEV):
            src = (my - d) % N_DEV
            krw = pltpu.make_async_remote_copy(
                src_ref=kf_ref.at[:, pl.ds(SKV_SH * src, SKV_SH), :, :],
                dst_ref=kf_ref.at[:, pl.ds(SKV_SH * src, SKV_SH), :, :],
                send_sem=k_send.at[src],
                recv_sem=k_recv.at[src],
                device_id=(src,),
                device_id_type=pl.DeviceIdType.MESH,
            )
            vrw = pltpu.make_async_remote_copy(
                src_ref=vf_ref.at[:, pl.ds(SKV_SH * src, SKV_SH), :, :],
                dst_ref=vf_ref.at[:, pl.ds(SKV_SH * src, SKV_SH), :, :],
                send_sem=v_send.at[src],
                recv_sem=v_recv.at[src],
                device_id=(src,),
                device_id_type=pl.DeviceIdType.MESH,
            )
            krw.wait_recv()
            vrw.wait_recv()

        k_full = kf_ref[...]
        v_full = vf_ref[...]
        scores = lax.dot_general(
            q, k_full,
            dimension_numbers=(((3,), (3,)), ((0, 2), (0, 2))),
            preferred_element_type=jnp.float32,
        ) * 0.125
        scores = jnp.where(mask[None, None, :, :], scores, -1e9)
        m = jnp.max(scores, axis=-1, keepdims=True)
        w = jnp.exp(scores - m)
        w = w / jnp.sum(w, axis=-1, keepdims=True)
        ctx = lax.dot_general(
            w, v_full,
            dimension_numbers=(((3,), (1,)), ((0, 1), (0, 2))),
            preferred_element_type=jnp.float32,
        )
        ctx = ctx.transpose(0, 2, 1, 3).reshape(B * SQ, H_LOC * DH)
        partial = jnp.dot(
            ctx, wo_ref[...], preferred_element_type=jnp.float32
        ).reshape(B, SQ, D_MODEL)

        p_ref[...] = partial
        pc = pltpu.make_async_copy(p_ref, ar_ref.at[my], loc_sem.at[0])
        pc.start()
        ar_rdmas = []
        for d in range(1, N_DEV):
            tgt = (my + d) % N_DEV
            r = pltpu.make_async_remote_copy(
                src_ref=p_ref,
                dst_ref=ar_ref.at[my],
                send_sem=ar_send.at[tgt],
                recv_sem=ar_recv.at[my],
                device_id=(tgt,),
                device_id_type=pl.DeviceIdType.MESH,
            )
            r.start()
            ar_rdmas.append(r)
        pc.wait()
        for d in range(1, N_DEV):
            src = (my - d) % N_DEV
            rw = pltpu.make_async_remote_copy(
                src_ref=ar_ref.at[src],
                dst_ref=ar_ref.at[src],
                send_sem=ar_send.at[src],
                recv_sem=ar_recv.at[src],
                device_id=(src,),
                device_id_type=pl.DeviceIdType.MESH,
            )
            rw.wait_recv()

        out_ref[...] = ar_ref[0] + ar_ref[1] + ar_ref[2] + ar_ref[3]

        for kr, vr in kv_rdmas:
            kr.wait_send()
            vr.wait_send()
        for r in ar_rdmas:
            r.wait_send()

    return pl.pallas_call(
        body,
        out_shape=jax.ShapeDtypeStruct((B, SQ, D_MODEL), jnp.float32),
        in_specs=[pl.BlockSpec(memory_space=pltpu.VMEM)] * 5,
        out_specs=pl.BlockSpec(memory_space=pltpu.VMEM),
        scratch_shapes=[
            pltpu.VMEM((B, SKV, H_LOC, DH), jnp.float32),
            pltpu.VMEM((B, SKV, H_LOC, DH), jnp.float32),
            pltpu.VMEM((N_DEV, B, SQ, D_MODEL), jnp.float32),
            pltpu.VMEM((B, SQ, D_MODEL), jnp.float32),
            pltpu.SemaphoreType.DMA((N_DEV,)),
            pltpu.SemaphoreType.DMA((N_DEV,)),
            pltpu.SemaphoreType.DMA((N_DEV,)),
            pltpu.SemaphoreType.DMA((N_DEV,)),
            pltpu.SemaphoreType.DMA((N_DEV,)),
            pltpu.SemaphoreType.DMA((N_DEV,)),
            pltpu.SemaphoreType.DMA((2,)),
        ],
        compiler_params=pltpu.CompilerParams(collective_id=0),
    )(x, Wq, K_ext, V_ext, Wo)


# baseline (device time: 88015 ns/iter reference)
import jax
import jax.numpy as jnp
from jax import lax
from jax.experimental import pallas as pl
from jax.experimental.pallas import tpu as pltpu

N_DEV = 4
B, SQ, SKV_SH, HQ, DH = 2, 256, 256, 16, 64
H_LOC = HQ // N_DEV
SKV = SKV_SH * N_DEV
D_MODEL = 512


def kernel(x, Wq, K_ext, V_ext, Wo):
    def body(x_ref, wq_ref, k_ref, v_ref, wo_ref, out_ref,
             kf_ref, vf_ref, ar_ref, p_ref,
             k_send, k_recv, v_send, v_recv, ar_send, ar_recv, loc_sem):
        my = lax.axis_index("i")

        bsem = pltpu.get_barrier_semaphore()
        for d in range(1, N_DEV):
            pl.semaphore_signal(
                bsem, inc=1,
                device_id=((my + d) % N_DEV,),
                device_id_type=pl.DeviceIdType.MESH,
            )
        pl.semaphore_wait(bsem, N_DEV - 1)

        kc = pltpu.make_async_copy(
            k_ref.at[:, :, pl.ds(H_LOC * my, H_LOC), :],
            kf_ref.at[:, pl.ds(SKV_SH * my, SKV_SH), :, :],
            loc_sem.at[0],
        )
        vc = pltpu.make_async_copy(
            v_ref.at[:, :, pl.ds(H_LOC * my, H_LOC), :],
            vf_ref.at[:, pl.ds(SKV_SH * my, SKV_SH), :, :],
            loc_sem.at[1],
        )
        kc.start()
        vc.start()

        kv_rdmas = []
        for d in range(1, N_DEV):
            tgt = (my + d) % N_DEV
            kr = pltpu.make_async_remote_copy(
                src_ref=k_ref.at[:, :, pl.ds(H_LOC * tgt, H_LOC), :],
                dst_ref=kf_ref.at[:, pl.ds(SKV_SH * my, SKV_SH), :, :],
                send_sem=k_send.at[tgt],
                recv_sem=k_recv.at[my],
                device_id=(tgt,),
                device_id_type=pl.DeviceIdType.MESH,
            )
            vr = pltpu.make_async_remote_copy(
                src_ref=v_ref.at[:, :, pl.ds(H_LOC * tgt, H_LOC), :],
                dst_ref=vf_ref.at[:, pl.ds(SKV_SH * my, SKV_SH), :, :],
                send_sem=v_send.at[tgt],
                recv_sem=v_recv.at[my],
                device_id=(tgt,),
                device_id_type=pl.DeviceIdType.MESH,
            )
            kr.start()
            vr.start()
            kv_rdmas.append((kr, vr))

        q2d = jnp.dot(
            x_ref[...].reshape(B * SQ, D_MODEL), wq_ref[...],
            preferred_element_type=jnp.float32,
        )
        q = q2d.reshape(B, SQ, H_LOC, DH)

        qi = lax.broadcasted_iota(jnp.int32, (SQ, SKV), 0)
        ki = lax.broadcasted_iota(jnp.int32, (SQ, SKV), 1)
        mask = (jnp.abs(qi - ki) <= 128) | (ki < 32) | (qi < 32)

        kc.wait()
        vc.wait()
        for d in range(1, N_DEV):
            src = (my - d) % N_DEV
            krw = pltpu.make_async_remote_copy(
                src_ref=kf_ref.at[:, pl.ds(SKV_SH * src, SKV_SH), :, :],
                dst_ref=kf_ref.at[:, pl.ds(SKV_SH * src, SKV_SH), :, :],
                send_sem=k_send.at[src],
                recv_sem=k_recv.at[src],
                device_id=(src,),
                device_id_type=pl.DeviceIdType.MESH,
            )
            vrw = pltpu.make_async_remote_copy(
                src_ref=vf_ref.at[:, pl.ds(SKV_SH * src, SKV_SH), :, :],
                dst_ref=vf_ref.at[:, pl.ds(SKV_SH * src, SKV_SH), :, :],
                send_sem=v_send.at[src],
                recv_sem=v_recv.at[src],
                device_id=(src,),
                device_id_type=pl.DeviceIdType.MESH,
            )
            krw.wait_recv()
            vrw.wait_recv()

        qb = q.transpose(0, 2, 1, 3).reshape(B * H_LOC, SQ, DH)
        kb = kf_ref[...].transpose(0, 2, 1, 3).reshape(B * H_LOC, SKV, DH)
        vb = vf_ref[...].transpose(0, 2, 1, 3).reshape(B * H_LOC, SKV, DH)
        scores = lax.dot_general(
            qb, kb,
            dimension_numbers=(((2,), (2,)), ((0,), (0,))),
            preferred_element_type=jnp.float32,
        ) * 0.125
        scores = jnp.where(mask[None, :, :], scores, -1e9)
        m = jnp.max(scores, axis=-1, keepdims=True)
        w = jnp.exp(scores - m)
        w = w / jnp.sum(w, axis=-1, keepdims=True)
        ctx = lax.dot_general(
            w, vb,
            dimension_numbers=(((2,), (1,)), ((0,), (0,))),
            preferred_element_type=jnp.float32,
        )
        ctx = ctx.reshape(B, H_LOC, SQ, DH).transpose(0, 2, 1, 3)
        ctx = ctx.reshape(B * SQ, H_LOC * DH)
        partial = jnp.dot(
            ctx, wo_ref[...], preferred_element_type=jnp.float32
        ).reshape(B, SQ, D_MODEL)

        p_ref[...] = partial
        pc = pltpu.make_async_copy(p_ref, ar_ref.at[my], loc_sem.at[0])
        pc.start()
        ar_rdmas = []
        for d in range(1, N_DEV):
            tgt = (my + d) % N_DEV
            r = pltpu.make_async_remote_copy(
                src_ref=p_ref,
                dst_ref=ar_ref.at[my],
                send_sem=ar_send.at[tgt],
                recv_sem=ar_recv.at[my],
                device_id=(tgt,),
                device_id_type=pl.DeviceIdType.MESH,
            )
            r.start()
            ar_rdmas.append(r)
        pc.wait()
        for d in range(1, N_DEV):
            src = (my - d) % N_DEV
            rw = pltpu.make_async_remote_copy(
                src_ref=ar_ref.at[src],
                dst_ref=ar_ref.at[src],
                send_sem=ar_send.at[src],
                recv_sem=ar_recv.at[src],
                device_id=(src,),
                device_id_type=pl.DeviceIdType.MESH,
            )
            rw.wait_recv()

        out_ref[...] = ar_ref[0] + ar_ref[1] + ar_ref[2] + ar_ref[3]

        for kr, vr in kv_rdmas:
            kr.wait_send()
            vr.wait_send()
        for r in ar_rdmas:
            r.wait_send()

    return pl.pallas_call(
        body,
        out_shape=jax.ShapeDtypeStruct((B, SQ, D_MODEL), jnp.float32),
        in_specs=[pl.BlockSpec(memory_space=pltpu.VMEM)] * 5,
        out_specs=pl.BlockSpec(memory_space=pltpu.VMEM),
        scratch_shapes=[
            pltpu.VMEM((B, SKV, H_LOC, DH), jnp.float32),
            pltpu.VMEM((B, SKV, H_LOC, DH), jnp.float32),
            pltpu.VMEM((N_DEV, B, SQ, D_MODEL), jnp.float32),
            pltpu.VMEM((B, SQ, D_MODEL), jnp.float32),
            pltpu.SemaphoreType.DMA((N_DEV,)),
            pltpu.SemaphoreType.DMA((N_DEV,)),
            pltpu.SemaphoreType.DMA((N_DEV,)),
            pltpu.SemaphoreType.DMA((N_DEV,)),
            pltpu.SemaphoreType.DMA((N_DEV,)),
            pltpu.SemaphoreType.DMA((N_DEV,)),
            pltpu.SemaphoreType.DMA((2,)),
        ],
        compiler_params=pltpu.CompilerParams(collective_id=0),
    )(x, Wq, K_ext, V_ext, Wo)


# device time: 48787 ns/iter; 1.8041x vs baseline; 1.8041x over previous
import jax
import jax.numpy as jnp
from jax import lax
from jax.experimental import pallas as pl
from jax.experimental.pallas import tpu as pltpu

N_DEV = 4
B, SQ, SKV_SH, HQ, DH = 2, 256, 256, 16, 64
H_LOC = HQ // N_DEV
SKV = SKV_SH * N_DEV
D_MODEL = 512
BH = B * H_LOC
ROWS = B * SQ
R_SH = ROWS // N_DEV


def kernel(x, Wq, K_ext, V_ext, Wo):
    def body(x_ref, wq_ref, k_ref, v_ref, wo_ref, out_ref,
             kvs_ref, kvf_ref, pb_ref, rs_ref, red_ref, ag_ref,
             kv_send, kv_recv, rs_send, rs_recv, ag_send, ag_recv, loc_sem):
        my = lax.axis_index("i")

        bsem = pltpu.get_barrier_semaphore()
        for d in range(1, N_DEV):
            pl.semaphore_signal(
                bsem, inc=1,
                device_id=((my + d) % N_DEV,),
                device_id_type=pl.DeviceIdType.MESH,
            )
        pl.semaphore_wait(bsem, N_DEV - 1)

        kvs_ref[0] = k_ref[...].astype(jnp.bfloat16)
        kvs_ref[1] = v_ref[...].astype(jnp.bfloat16)

        lc = pltpu.make_async_copy(
            kvs_ref.at[:, :, :, pl.ds(H_LOC * my, H_LOC), :],
            kvf_ref.at[0],
            loc_sem.at[0],
        )
        lc.start()

        kv_rdmas = []
        for d in range(1, N_DEV):
            tgt = (my + d) % N_DEV
            r = pltpu.make_async_remote_copy(
                src_ref=kvs_ref.at[:, :, :, pl.ds(H_LOC * tgt, H_LOC), :],
                dst_ref=kvf_ref.at[d],
                send_sem=kv_send.at[d],
                recv_sem=kv_recv.at[d],
                device_id=(tgt,),
                device_id_type=pl.DeviceIdType.MESH,
            )
            r.start()
            kv_rdmas.append(r)

        qb = jnp.dot(
            x_ref[...].reshape(ROWS, D_MODEL).astype(jnp.bfloat16),
            wq_ref[...].astype(jnp.bfloat16),
            preferred_element_type=jnp.float32,
        )
        qb = (
            qb.reshape(B, SQ, H_LOC, DH)
            .transpose(0, 2, 1, 3)
            .reshape(BH, SQ, DH)
            .astype(jnp.bfloat16)
        )

        qi = lax.broadcasted_iota(jnp.int32, (SQ, SKV_SH), 0)
        kij = lax.broadcasted_iota(jnp.int32, (SQ, SKV_SH), 1)

        acc = jnp.zeros((BH, SQ, DH), jnp.float32)
        lsum = jnp.zeros((BH, SQ), jnp.float32)
        for d in range(N_DEV):
            if d == 0:
                lc.wait()
            else:
                w = pltpu.make_async_remote_copy(
                    src_ref=kvf_ref.at[d],
                    dst_ref=kvf_ref.at[d],
                    send_sem=kv_send.at[d],
                    recv_sem=kv_recv.at[d],
                    device_id=((my - d) % N_DEV,),
                    device_id_type=pl.DeviceIdType.MESH,
                )
                w.wait_recv()
            src = (my - d) % N_DEV
            kv = kvf_ref[d]
            kb = kv[0].transpose(0, 2, 1, 3).reshape(BH, SKV_SH, DH)
            vb = kv[1].transpose(0, 2, 1, 3).reshape(BH, SKV_SH, DH)
            s = lax.dot_general(
                qb, kb,
                dimension_numbers=(((2,), (2,)), ((0,), (0,))),
                preferred_element_type=jnp.float32,
            ) * 0.125
            ki = kij + src * SKV_SH
            mask = (jnp.abs(qi - ki) <= 128) | (ki < 32) | (qi < 32)
            p = jnp.where(mask[None, :, :], jnp.exp(s), 0.0)
            lsum = lsum + jnp.sum(p, axis=-1)
            acc = acc + lax.dot_general(
                p.astype(jnp.bfloat16), vb,
                dimension_numbers=(((2,), (1,)), ((0,), (0,))),
                preferred_element_type=jnp.float32,
            )

        ctx = acc / lsum[:, :, None]
        ctx = (
            ctx.reshape(B, H_LOC, SQ, DH)
            .transpose(0, 2, 1, 3)
            .reshape(ROWS, H_LOC * DH)
            .astype(jnp.bfloat16)
        )
        partial = jnp.dot(
            ctx, wo_ref[...].astype(jnp.bfloat16),
            preferred_element_type=jnp.float32,
        )
        pb_ref[...] = partial.astype(jnp.bfloat16)

        rs_rdmas = []
        for d in range(1, N_DEV):
            tgt = (my + d) % N_DEV
            r = pltpu.make_async_remote_copy(
                src_ref=pb_ref.at[pl.ds(R_SH * tgt, R_SH), :],
                dst_ref=rs_ref.at[d],
                send_sem=rs_send.at[d],
                recv_sem=rs_recv.at[d],
                device_id=(tgt,),
                device_id_type=pl.DeviceIdType.MESH,
            )
            r.start()
            rs_rdmas.append(r)
        for d in range(1, N_DEV):
            w = pltpu.make_async_remote_copy(
                src_ref=rs_ref.at[d],
                dst_ref=rs_ref.at[d],
                send_sem=rs_send.at[d],
                recv_sem=rs_recv.at[d],
                device_id=((my - d) % N_DEV,),
                device_id_type=pl.DeviceIdType.MESH,
            )
            w.wait_recv()
        red = pb_ref[pl.ds(R_SH * my, R_SH), :].astype(jnp.float32)
        for d in range(1, N_DEV):
            red = red + rs_ref[d].astype(jnp.float32)

        red_ref[...] = red.astype(jnp.bfloat16)
        ag_rdmas = []
        for d in range(1, N_DEV):
            tgt = (my + d) % N_DEV
            r = pltpu.make_async_remote_copy(
                src_ref=red_ref,
                dst_ref=ag_ref.at[d],
                send_sem=ag_send.at[d],
                recv_sem=ag_recv.at[d],
                device_id=(tgt,),
                device_id_type=pl.DeviceIdType.MESH,
            )
            r.start()
            ag_rdmas.append(r)

        my_b = my // 2
        my_off = (my % 2) * R_SH
        out_ref[my_b, pl.ds(my_off, R_SH), :] = red

        for d in range(1, N_DEV):
            w = pltpu.make_async_remote_copy(
                src_ref=ag_ref.at[d],
                dst_ref=ag_ref.at[d],
                send_sem=ag_send.at[d],
                recv_sem=ag_recv.at[d],
                device_id=((my - d) % N_DEV,),
                device_id_type=pl.DeviceIdType.MESH,
            )
            w.wait_recv()
            src = (my - d) % N_DEV
            out_ref[src // 2, pl.ds((src % 2) * R_SH, R_SH), :] = (
                ag_ref[d].astype(jnp.float32)
            )

        for r in kv_rdmas + rs_rdmas + ag_rdmas:
            r.wait_send()

    return pl.pallas_call(
        body,
        out_shape=jax.ShapeDtypeStruct((B, SQ, D_MODEL), jnp.float32),
        in_specs=[pl.BlockSpec(memory_space=pltpu.VMEM)] * 5,
        out_specs=pl.BlockSpec(memory_space=pltpu.VMEM),
        scratch_shapes=[
            pltpu.VMEM((2, B, SKV_SH, HQ, DH), jnp.bfloat16),
            pltpu.VMEM((N_DEV, 2, B, SKV_SH, H_LOC, DH), jnp.bfloat16),
            pltpu.VMEM((ROWS, D_MODEL), jnp.bfloat16),
            pltpu.VMEM((N_DEV, R_SH, D_MODEL), jnp.bfloat16),
            pltpu.VMEM((R_SH, D_MODEL), jnp.bfloat16),
            pltpu.VMEM((N_DEV, R_SH, D_MODEL), jnp.bfloat16),
            pltpu.SemaphoreType.DMA((N_DEV,)),
            pltpu.SemaphoreType.DMA((N_DEV,)),
            pltpu.SemaphoreType.DMA((N_DEV,)),
            pltpu.SemaphoreType.DMA((N_DEV,)),
            pltpu.SemaphoreType.DMA((N_DEV,)),
            pltpu.SemaphoreType.DMA((N_DEV,)),
            pltpu.SemaphoreType.DMA((2,)),
        ],
        compiler_params=pltpu.CompilerParams(collective_id=0),
    )(x, Wq, K_ext, V_ext, Wo)
